# baseline (speedup 1.0000x reference)
import functools

import jax
import jax.numpy as jnp
from jax import lax
from jax.experimental import pallas as pl
from jax.experimental.pallas import tpu as pltpu
from jax.experimental.pallas import tpu_sc as plsc

N = 10000
E = 320000
N_PAD = 10240
RPT = N_PAD // 16
CHUNK = 128
CPW = 80
E_PAD = 32 * CPW * CHUNK


@functools.cache
def _mesh():
    return plsc.VectorSubcoreMesh(
        core_axis_name="c", subcore_axis_name="s", num_cores=2, num_subcores=16
    )


@functools.cache
def _deg_kernel():
    return functools.partial(
        pl.kernel,
        out_type=jax.ShapeDtypeStruct((2, N_PAD), jnp.float32),
        mesh=_mesh(),
        compiler_params=pltpu.CompilerParams(use_tc_tiling_on_sc=False),
        scratch_types=[
            pltpu.VMEM((CPW, CHUNK), jnp.int32),
            pltpu.VMEM((CHUNK,), jnp.float32),
            pltpu.VMEM_SHARED((N_PAD,), jnp.float32),
        ],
    )(_deg_body)


def _deg_body(dst32_hbm, zeros1_hbm, out_hbm, idx_v, ones_v, acc_sh):
    c = lax.axis_index("c")
    s = lax.axis_index("s")
    wid = s * 2 + c
    pltpu.sync_copy(zeros1_hbm.at[pl.ds(s * RPT, RPT)],
                    acc_sh.at[pl.ds(s * RPT, RPT)])
    for i in range(CHUNK // 16):
        ones_v[pl.ds(i * 16, 16)] = jnp.full((16,), 1.0, jnp.float32)
    pltpu.sync_copy(dst32_hbm.at[wid], idx_v)
    plsc.subcore_barrier()

    def body(j, carry):
        pltpu.sync_copy(ones_v, acc_sh.at[idx_v.at[j]], add=True)
        return carry
    lax.fori_loop(0, CPW, body, 0)

    plsc.subcore_barrier()
    pltpu.sync_copy(acc_sh.at[pl.ds(s * RPT, RPT)],
                    out_hbm.at[c, pl.ds(s * RPT, RPT)])


@functools.cache
def _make_prop(C):
    if C == 1:
        row_shape, acc_shape, out_shape = (CHUNK,), (N_PAD,), (2, N_PAD)
    else:
        row_shape, acc_shape, out_shape = (CHUNK, C), (N_PAD, C), (2, N_PAD, C)

    @functools.partial(
        pl.kernel,
        out_type=jax.ShapeDtypeStruct(out_shape, jnp.float32),
        mesh=_mesh(),
        compiler_params=pltpu.CompilerParams(use_tc_tiling_on_sc=False),
        scratch_types=[
            pltpu.VMEM((CPW, CHUNK), jnp.int32),
            pltpu.VMEM((CPW, CHUNK), jnp.int32),
            pltpu.VMEM(row_shape, jnp.float32),
            pltpu.VMEM_SHARED(acc_shape, jnp.float32),
            pltpu.SemaphoreType.DMA,
        ],
    )
    def prop(g_hbm, zeros_hbm, src_hbm, dst_hbm, out_hbm,
             src_v, dst_v, rows_v, acc_sh, sem):
        c = lax.axis_index("c")
        s = lax.axis_index("s")
        wid = s * 2 + c

        @pl.when(c == 0)
        def _():
            pltpu.sync_copy(g_hbm.at[pl.ds(s * RPT, RPT)],
                            acc_sh.at[pl.ds(s * RPT, RPT)])

        @pl.when(c == 1)
        def _():
            pltpu.sync_copy(zeros_hbm.at[pl.ds(s * RPT, RPT)],
                            acc_sh.at[pl.ds(s * RPT, RPT)])

        pltpu.sync_copy(src_hbm.at[wid], src_v)
        pltpu.sync_copy(dst_hbm.at[wid], dst_v)
        plsc.subcore_barrier()

        def body(j, carry):
            pltpu.async_copy(g_hbm.at[src_v.at[j]], rows_v, sem).wait()
            pltpu.sync_copy(rows_v, acc_sh.at[dst_v.at[j]], add=True)
            return carry
        lax.fori_loop(0, CPW, body, 0)

        plsc.subcore_barrier()
        pltpu.sync_copy(acc_sh.at[pl.ds(s * RPT, RPT)],
                        out_hbm.at[c, pl.ds(s * RPT, RPT)])

    return prop


_BLK = 1024
_GRID = N_PAD // _BLK


def _k1_body(x_ref, w1_ref, deg_ref, g1_ref, dinv_ref):
    deg = deg_ref[0] + deg_ref[1]
    dinv = lax.rsqrt(deg + 1.0)
    h = jnp.dot(x_ref[...], w1_ref[...], preferred_element_type=jnp.float32)
    g1_ref[...] = h * dinv
    dinv_ref[...] = dinv


def _k2_body(acc_ref, dinv_ref, b1_ref, w2_ref, g2_ref):
    pre = acc_ref[0] + acc_ref[1]
    dinv = dinv_ref[...]
    h1 = jnp.maximum(pre * dinv + b1_ref[...], 0.0)
    g2_ref[...] = jnp.dot(h1, w2_ref[...],
                          preferred_element_type=jnp.float32) * dinv


def _k3_body(acc_ref, dinv_ref, b2_ref, w3_ref, g3_ref):
    pre = acc_ref[0] + acc_ref[1]
    dinv = dinv_ref[...]
    h2 = jnp.maximum(pre * dinv + b2_ref[...], 0.0)
    g3_ref[...] = jnp.dot(h2, w3_ref[...],
                          preferred_element_type=jnp.float32) * dinv


def _k4_body(acc3_ref, dinv_ref, b3_ref, wfc_ref, bfc_ref, out_ref):
    pre = acc3_ref[0] + acc3_ref[1]
    h3 = pre * dinv_ref[...] + b3_ref[...]
    out_ref[...] = h3 * wfc_ref[...] + bfc_ref[...]


def _col_spec():
    return pl.BlockSpec((_BLK, 1), lambda i: (i, 0))


def _col2_spec():
    return pl.BlockSpec((2, _BLK, 1), lambda i: (0, i, 0))


def _full_spec(shape):
    nd = len(shape)
    return pl.BlockSpec(shape, lambda i: (0,) * nd)


def _tc_k1(x_pad, W1, deg2):
    return pl.pallas_call(
        _k1_body,
        grid=(_GRID,),
        in_specs=[
            pl.BlockSpec((_BLK, 128), lambda i: (i, 0)),
            _full_spec((128, 128)),
            _col2_spec(),
        ],
        out_specs=[
            pl.BlockSpec((_BLK, 128), lambda i: (i, 0)),
            _col_spec(),
        ],
        out_shape=[
            jax.ShapeDtypeStruct((N_PAD, 128), jnp.float32),
            jax.ShapeDtypeStruct((N_PAD, 1), jnp.float32),
        ],
    )(x_pad, W1, deg2)


def _tc_k2(acc1, dinv_col, b1, W2):
    return pl.pallas_call(
        _k2_body,
        grid=(_GRID,),
        in_specs=[
            pl.BlockSpec((2, _BLK, 128), lambda i: (0, i, 0)),
            _col_spec(),
            _full_spec((1, 128)),
            _full_spec((128, 64)),
        ],
        out_specs=pl.BlockSpec((_BLK, 64), lambda i: (i, 0)),
        out_shape=jax.ShapeDtypeStruct((N_PAD, 64), jnp.float32),
    )(acc1, dinv_col, b1, W2)


def _tc_k3(acc2, dinv_col, b2, W3):
    return pl.pallas_call(
        _k3_body,
        grid=(_GRID,),
        in_specs=[
            pl.BlockSpec((2, _BLK, 64), lambda i: (0, i, 0)),
            _col_spec(),
            _full_spec((1, 64)),
            _full_spec((64, 1)),
        ],
        out_specs=_col_spec(),
        out_shape=jax.ShapeDtypeStruct((N_PAD, 1), jnp.float32),
    )(acc2, dinv_col, b2, W3)


def _tc_k4(acc3, dinv_col, b3, Wfc, bfc):
    return pl.pallas_call(
        _k4_body,
        grid=(_GRID,),
        in_specs=[
            _col2_spec(),
            _col_spec(),
            _full_spec((1, 1)),
            _full_spec((1, 16)),
            _full_spec((1, 16)),
        ],
        out_specs=pl.BlockSpec((_BLK, 16), lambda i: (i, 0)),
        out_shape=jax.ShapeDtypeStruct((N_PAD, 16), jnp.float32),
    )(acc3, dinv_col, b3, Wfc, bfc)


@jax.jit
def _run(x, edge_index, W1, b1, W2, b2, W3, b3, Wfc, bfc):
    src = edge_index[0]
    dst = edge_index[1]

    pad_dst = N + jnp.arange(E_PAD - E, dtype=jnp.int32) % (N_PAD - N)
    src32 = jnp.full((E_PAD,), 0, jnp.int32).at[:E].set(src).reshape(32, CPW, CHUNK)
    dst32 = (jnp.zeros((E_PAD,), jnp.int32).at[:E].set(dst)
             .at[E:].set(pad_dst).reshape(32, CPW, CHUNK))

    x_pad = jnp.zeros((N_PAD, 128), jnp.float32).at[:N].set(x)
    zeros1 = jnp.zeros((N_PAD,), jnp.float32)
    zeros128 = jnp.zeros((N_PAD, 128), jnp.float32)
    zeros64 = jnp.zeros((N_PAD, 64), jnp.float32)

    deg2 = _deg_kernel()(dst32, zeros1)

    g1, dinv_col = _tc_k1(x_pad, W1, deg2.reshape(2, N_PAD, 1))
    acc1 = _make_prop(128)(g1, zeros128, src32, dst32)
    g2 = _tc_k2(acc1, dinv_col, b1.reshape(1, 128), W2)
    acc2 = _make_prop(64)(g2, zeros64, src32, dst32)
    g3_col = _tc_k3(acc2, dinv_col, b2.reshape(1, 64), W3)
    acc3 = _make_prop(1)(g3_col.reshape(N_PAD), zeros1, src32, dst32)
    out = _tc_k4(acc3.reshape(2, N_PAD, 1), dinv_col,
                 b3.reshape(1, 1), Wfc, bfc.reshape(1, 16))
    return out[:N]


def kernel(x, edge_index, W1, b1, W2, b2, W3, b3, Wfc, bfc):
    return _run(x, edge_index, W1, b1, W2, b2, W3, b3, Wfc, bfc)


# speedup vs baseline: 2.1718x; 2.1718x over previous
import functools

import jax
import jax.numpy as jnp
from jax import lax
from jax.experimental import pallas as pl
from jax.experimental.pallas import tpu as pltpu
from jax.experimental.pallas import tpu_sc as plsc

N = 10000
E = 320000
N_PAD = 10240
RPT = N_PAD // 16
CHUNK = 128
CPW = 80
E_PAD = 32 * CPW * CHUNK
CPW16 = 160
HALF16 = CPW16 // 2


@functools.cache
def _mesh():
    return plsc.VectorSubcoreMesh(
        core_axis_name="c", subcore_axis_name="s", num_cores=2, num_subcores=16
    )


_SC_PARAMS = pltpu.CompilerParams(use_tc_tiling_on_sc=False)


@functools.cache
def _deg_kernel():
    return functools.partial(
        pl.kernel,
        out_type=jax.ShapeDtypeStruct((2, N_PAD), jnp.float32),
        mesh=_mesh(),
        compiler_params=_SC_PARAMS,
        scratch_types=[
            pltpu.VMEM((CPW, CHUNK), jnp.int32),
            pltpu.VMEM((CHUNK,), jnp.float32),
            pltpu.VMEM_SHARED((N_PAD,), jnp.float32),
        ],
    )(_deg_body)


def _deg_body(dst32_hbm, zeros1_hbm, out_hbm, idx_v, ones_v, acc_sh):
    c = lax.axis_index("c")
    s = lax.axis_index("s")
    wid = s * 2 + c
    pltpu.sync_copy(zeros1_hbm.at[pl.ds(s * RPT, RPT)],
                    acc_sh.at[pl.ds(s * RPT, RPT)])
    for i in range(CHUNK // 16):
        ones_v[pl.ds(i * 16, 16)] = jnp.full((16,), 1.0, jnp.float32)
    pltpu.sync_copy(dst32_hbm.at[wid], idx_v)
    plsc.subcore_barrier()

    def body(j, carry):
        pltpu.sync_copy(ones_v, acc_sh.at[idx_v.at[j]], add=True)
        return carry
    lax.fori_loop(0, CPW, body, 0)

    plsc.subcore_barrier()
    pltpu.sync_copy(acc_sh.at[pl.ds(s * RPT, RPT)],
                    out_hbm.at[c, pl.ds(s * RPT, RPT)])


@functools.cache
def _prop128cs_kernel():
    C = 64
    return functools.partial(
        pl.kernel,
        out_type=jax.ShapeDtypeStruct((2, N_PAD, C), jnp.float32),
        mesh=_mesh(),
        compiler_params=_SC_PARAMS,
        scratch_types=[
            pltpu.VMEM((HALF16, CHUNK), jnp.int32),
            pltpu.VMEM((HALF16, CHUNK), jnp.int32),
            pltpu.VMEM((CHUNK, C), jnp.float32),
            pltpu.VMEM_SHARED((N_PAD, C), jnp.float32),
            pltpu.VMEM_SHARED((N_PAD, C), jnp.float32),
            pltpu.SemaphoreType.DMA,
        ],
    )(_prop128cs_body)


def _prop128cs_body(g_hbm, src16_hbm, dst16_hbm, out_hbm,
                    src_v, dst_v, rows_v, g_sh, acc_sh, sem):
    c = lax.axis_index("c")
    s = lax.axis_index("s")
    sl = pl.ds(s * RPT, RPT)
    pltpu.sync_copy(g_hbm.at[c, sl], g_sh.at[sl])
    pltpu.sync_copy(g_hbm.at[c, sl], acc_sh.at[sl])
    plsc.subcore_barrier()

    for h in range(2):
        pltpu.sync_copy(src16_hbm.at[s, pl.ds(h * HALF16, HALF16)], src_v)
        pltpu.sync_copy(dst16_hbm.at[s, pl.ds(h * HALF16, HALF16)], dst_v)

        def body(j, carry):
            pltpu.async_copy(g_sh.at[src_v.at[j]], rows_v, sem).wait()
            pltpu.sync_copy(rows_v, acc_sh.at[dst_v.at[j]], add=True)
            return carry
        lax.fori_loop(0, HALF16, body, 0)

    plsc.subcore_barrier()
    pltpu.sync_copy(acc_sh.at[sl], out_hbm.at[c, sl])


@functools.cache
def _make_prop_es(C):
    if C == 1:
        row_shape, tab_shape, out_shape = (CHUNK,), (N_PAD,), (2, N_PAD)
    else:
        row_shape, tab_shape, out_shape = (CHUNK, C), (N_PAD, C), (2, N_PAD, C)

    @functools.partial(
        pl.kernel,
        out_type=jax.ShapeDtypeStruct(out_shape, jnp.float32),
        mesh=_mesh(),
        compiler_params=_SC_PARAMS,
        scratch_types=[
            pltpu.VMEM((CPW, CHUNK), jnp.int32),
            pltpu.VMEM((CPW, CHUNK), jnp.int32),
            pltpu.VMEM(row_shape, jnp.float32),
            pltpu.VMEM_SHARED(tab_shape, jnp.float32),
            pltpu.VMEM_SHARED(tab_shape, jnp.float32),
            pltpu.SemaphoreType.DMA,
        ],
    )
    def prop(g_hbm, zeros_hbm, src_hbm, dst_hbm, out_hbm,
             src_v, dst_v, rows_v, g_sh, acc_sh, sem):
        c = lax.axis_index("c")
        s = lax.axis_index("s")
        wid = s * 2 + c
        sl = pl.ds(s * RPT, RPT)

        pltpu.sync_copy(g_hbm.at[sl], g_sh.at[sl])

        @pl.when(c == 0)
        def _():
            pltpu.sync_copy(g_hbm.at[sl], acc_sh.at[sl])

        @pl.when(c == 1)
        def _():
            pltpu.sync_copy(zeros_hbm.at[sl], acc_sh.at[sl])

        pltpu.sync_copy(src_hbm.at[wid], src_v)
        pltpu.sync_copy(dst_hbm.at[wid], dst_v)
        plsc.subcore_barrier()

        def body(j, carry):
            pltpu.async_copy(g_sh.at[src_v.at[j]], rows_v, sem).wait()
            pltpu.sync_copy(rows_v, acc_sh.at[dst_v.at[j]], add=True)
            return carry
        lax.fori_loop(0, CPW, body, 0)

        plsc.subcore_barrier()
        pltpu.sync_copy(acc_sh.at[sl], out_hbm.at[c, sl])

    return prop


_BLK = 1024
_GRID = N_PAD // _BLK


def _k1_body(x_ref, w1_ref, deg_ref, g1_ref, dinv_ref):
    deg = deg_ref[0] + deg_ref[1]
    dinv = lax.rsqrt(deg + 1.0)
    h = jnp.dot(x_ref[...], w1_ref[...], preferred_element_type=jnp.float32)
    g = h * dinv
    g1_ref[0] = g[:, :64]
    g1_ref[1] = g[:, 64:]
    dinv_ref[...] = dinv


def _k2_body(acc_ref, dinv_ref, b1_ref, w2_ref, g2_ref):
    pre = jnp.concatenate([acc_ref[0], acc_ref[1]], axis=-1)
    dinv = dinv_ref[...]
    h1 = jnp.maximum(pre * dinv + b1_ref[...], 0.0)
    g2_ref[...] = jnp.dot(h1, w2_ref[...],
                          preferred_element_type=jnp.float32) * dinv


def _k3_body(acc_ref, dinv_ref, b2_ref, w3_ref, g3_ref):
    pre = acc_ref[0] + acc_ref[1]
    dinv = dinv_ref[...]
    h2 = jnp.maximum(pre * dinv + b2_ref[...], 0.0)
    g3_ref[...] = jnp.dot(h2, w3_ref[...],
                          preferred_element_type=jnp.float32) * dinv


def _k4_body(acc3_ref, dinv_ref, b3_ref, wfc_ref, bfc_ref, out_ref):
    pre = acc3_ref[0] + acc3_ref[1]
    h3 = pre * dinv_ref[...] + b3_ref[...]
    out_ref[...] = h3 * wfc_ref[...] + bfc_ref[...]


def _col_spec():
    return pl.BlockSpec((_BLK, 1), lambda i: (i, 0))


def _col2_spec():
    return pl.BlockSpec((2, _BLK, 1), lambda i: (0, i, 0))


def _full_spec(shape):
    nd = len(shape)
    return pl.BlockSpec(shape, lambda i: (0,) * nd)


def _tc_k1(x_pad, W1, deg2):
    return pl.pallas_call(
        _k1_body,
        grid=(_GRID,),
        in_specs=[
            pl.BlockSpec((_BLK, 128), lambda i: (i, 0)),
            _full_spec((128, 128)),
            _col2_spec(),
        ],
        out_specs=[
            pl.BlockSpec((2, _BLK, 64), lambda i: (0, i, 0)),
            _col_spec(),
        ],
        out_shape=[
            jax.ShapeDtypeStruct((2, N_PAD, 64), jnp.float32),
            jax.ShapeDtypeStruct((N_PAD, 1), jnp.float32),
        ],
    )(x_pad, W1, deg2)


def _tc_k2(acc1, dinv_col, b1, W2):
    return pl.pallas_call(
        _k2_body,
        grid=(_GRID,),
        in_specs=[
            pl.BlockSpec((2, _BLK, 64), lambda i: (0, i, 0)),
            _col_spec(),
            _full_spec((1, 128)),
            _full_spec((128, 64)),
        ],
        out_specs=pl.BlockSpec((_BLK, 64), lambda i: (i, 0)),
        out_shape=jax.ShapeDtypeStruct((N_PAD, 64), jnp.float32),
    )(acc1, dinv_col, b1, W2)


def _tc_k3(acc2, dinv_col, b2, W3):
    return pl.pallas_call(
        _k3_body,
        grid=(_GRID,),
        in_specs=[
            pl.BlockSpec((2, _BLK, 64), lambda i: (0, i, 0)),
            _col_spec(),
            _full_spec((1, 64)),
            _full_spec((64, 1)),
        ],
        out_specs=_col_spec(),
        out_shape=jax.ShapeDtypeStruct((N_PAD, 1), jnp.float32),
    )(acc2, dinv_col, b2, W3)


def _tc_k4(acc3, dinv_col, b3, Wfc, bfc):
    return pl.pallas_call(
        _k4_body,
        grid=(_GRID,),
        in_specs=[
            _col2_spec(),
            _col_spec(),
            _full_spec((1, 1)),
            _full_spec((1, 16)),
            _full_spec((1, 16)),
        ],
        out_specs=pl.BlockSpec((_BLK, 16), lambda i: (i, 0)),
        out_shape=jax.ShapeDtypeStruct((N_PAD, 16), jnp.float32),
    )(acc3, dinv_col, b3, Wfc, bfc)


@jax.jit
def _run(x, edge_index, W1, b1, W2, b2, W3, b3, Wfc, bfc):
    src = edge_index[0]
    dst = edge_index[1]

    pad_dst = N + jnp.arange(E_PAD - E, dtype=jnp.int32) % (N_PAD - N)
    src_flat = jnp.full((E_PAD,), 0, jnp.int32).at[:E].set(src)
    dst_flat = (jnp.zeros((E_PAD,), jnp.int32).at[:E].set(dst)
                .at[E:].set(pad_dst))
    src32 = src_flat.reshape(32, CPW, CHUNK)
    dst32 = dst_flat.reshape(32, CPW, CHUNK)
    src16 = src_flat.reshape(16, CPW16, CHUNK)
    dst16 = dst_flat.reshape(16, CPW16, CHUNK)

    x_pad = jnp.zeros((N_PAD, 128), jnp.float32).at[:N].set(x)
    zeros1 = jnp.zeros((N_PAD,), jnp.float32)
    zeros64 = jnp.zeros((N_PAD, 64), jnp.float32)

    deg2 = _deg_kernel()(dst32, zeros1)

    g1s, dinv_col = _tc_k1(x_pad, W1, deg2.reshape(2, N_PAD, 1))
    acc1 = _prop128cs_kernel()(g1s, src16, dst16)
    g2 = _tc_k2(acc1, dinv_col, b1.reshape(1, 128), W2)
    acc2 = _make_prop_es(64)(g2, zeros64, src32, dst32)
    g3_col = _tc_k3(acc2, dinv_col, b2.reshape(1, 64), W3)
    acc3 = _make_prop_es(1)(g3_col.reshape(N_PAD), zeros1, src32, dst32)
    out = _tc_k4(acc3.reshape(2, N_PAD, 1), dinv_col,
                 b3.reshape(1, 1), Wfc, bfc.reshape(1, 16))
    return out[:N]


def kernel(x, edge_index, W1, b1, W2, b2, W3, b3, Wfc, bfc):
    return _run(x, edge_index, W1, b1, W2, b2, W3, b3, Wfc, bfc)
